# baseline (speedup 1.0000x reference)
import jax
import jax.numpy as jnp
from jax.experimental import pallas as pl


def _conv_nhwc(x, w, b, stride, pad):
    y = jax.lax.conv_general_dilated(x, w, (stride, stride), [(pad, pad), (pad, pad)],
                                     dimension_numbers=('NHWC', 'OIHW', 'NHWC'))
    return y + b[None, None, None, :]


def _convT_nhwc(x, w, b):
    y = jax.lax.conv_transpose(x, w, (2, 2), 'SAME', dimension_numbers=('NHWC', 'OIHW', 'NHWC'))
    return y + b[None, None, None, :]


_HT = 16


def _vq_kernel(z_ref, cb_ref, zq_ref, loss_ref):
    C = z_ref.shape[3]
    n = z_ref.shape[1] * z_ref.shape[2]
    z = z_ref[0].reshape(n, C)
    cb = cb_ref[...]
    cnorm = jnp.sum(cb * cb, axis=1)
    scores = jax.lax.dot_general(
        cb.astype(jnp.bfloat16), z.astype(jnp.bfloat16), (((1,), (1,)), ((), ())),
        preferred_element_type=jnp.float32)
    d2 = cnorm[:, None] - 2.0 * scores
    idx = jnp.argmin(d2, axis=0)

    onehot = (jax.lax.broadcasted_iota(jnp.int32, d2.shape, 0)
              == idx[None, :]).astype(jnp.float32)
    zq = jax.lax.dot_general(
        onehot, cb, (((0,), (0,)), ((), ())),
        precision=jax.lax.Precision.HIGHEST,
        preferred_element_type=jnp.float32)
    zq_ref[...] = zq.reshape(z_ref.shape)
    diff = z - zq
    part = jnp.sum(diff * diff).reshape(1, 1)

    @pl.when((pl.program_id(0) == 0) & (pl.program_id(1) == 0))
    def _():
        loss_ref[...] = jnp.zeros_like(loss_ref)
    loss_ref[...] += part


def _vq(z_e, codebook):
    B, H, W, C = z_e.shape
    zq, loss_sum = pl.pallas_call(
        _vq_kernel,
        grid=(B, H // _HT),
        in_specs=[
            pl.BlockSpec((1, _HT, W, C), lambda i, j: (i, j, 0, 0)),
            pl.BlockSpec(codebook.shape, lambda i, j: (0, 0)),
        ],
        out_specs=[
            pl.BlockSpec((1, _HT, W, C), lambda i, j: (i, j, 0, 0)),
            pl.BlockSpec((1, 1), lambda i, j: (0, 0)),
        ],
        out_shape=[
            jax.ShapeDtypeStruct(z_e.shape, jnp.float32),
            jax.ShapeDtypeStruct((1, 1), jnp.float32),
        ],
    )(z_e, codebook)
    mean_sq = loss_sum[0, 0] / (B * C * H * W)
    return zq, mean_sq


def kernel(x, enc_w1, enc_b1, enc_w2, enc_b2, enc_w3, enc_b3, codebook,
           dec_w1, dec_b1, dec_w2, dec_b2, dec_w3, dec_b3):
    commitment_cost = 0.25
    xh = jnp.transpose(x, (0, 2, 3, 1))
    h = jax.nn.relu(_conv_nhwc(xh, enc_w1, enc_b1, 2, 1))
    h = jax.nn.relu(_conv_nhwc(h, enc_w2, enc_b2, 2, 1))
    z_e = _conv_nhwc(h, enc_w3, enc_b3, 1, 1)

    z_q, mean_sq = _vq(z_e, codebook)
    commitment_loss = mean_sq
    codebook_loss = mean_sq
    vq_loss = codebook_loss + commitment_cost * commitment_loss

    g = jax.nn.relu(_conv_nhwc(z_q, dec_w1, dec_b1, 1, 1))
    g = jax.nn.relu(_convT_nhwc(g, dec_w2, dec_b2))
    x_recon = _convT_nhwc(g, dec_w3, dec_b3)
    return (jnp.transpose(x_recon, (0, 3, 1, 2)), vq_loss, commitment_loss, codebook_loss)


# speedup vs baseline: 1.0654x; 1.0654x over previous
import jax
import jax.numpy as jnp
from jax.experimental import pallas as pl


def _conv_nhwc(x, w, b, stride, pad):
    y = jax.lax.conv_general_dilated(x, w, (stride, stride), [(pad, pad), (pad, pad)],
                                     dimension_numbers=('NHWC', 'OIHW', 'NHWC'))
    return y + b[None, None, None, :]


def _convT_nhwc(x, w, b):
    y = jax.lax.conv_transpose(x, w, (2, 2), 'SAME', dimension_numbers=('NHWC', 'OIHW', 'NHWC'))
    return y + b[None, None, None, :]


_HT = 16


def _vq_kernel(z_ref, cb_ref, zq_ref, loss_ref):
    C = z_ref.shape[3]
    n = z_ref.shape[1] * z_ref.shape[2]
    z = z_ref[0].reshape(n, C)
    cb = cb_ref[...]
    cnorm = jnp.sum(cb * cb, axis=1)
    scores = jax.lax.dot_general(
        cb.astype(jnp.bfloat16), z.astype(jnp.bfloat16), (((1,), (1,)), ((), ())),
        preferred_element_type=jnp.float32)
    d2 = cnorm[:, None] - 2.0 * scores
    idx = jnp.argmin(d2, axis=0)

    onehot = (jax.lax.broadcasted_iota(jnp.int32, d2.shape, 0)
              == idx[None, :]).astype(jnp.bfloat16)
    zq = jax.lax.dot_general(
        onehot, cb.astype(jnp.bfloat16), (((0,), (0,)), ((), ())),
        preferred_element_type=jnp.float32)
    zq_ref[...] = zq.reshape(z_ref.shape)
    diff = z - zq
    part = jnp.sum(diff * diff).reshape(1, 1)

    @pl.when((pl.program_id(0) == 0) & (pl.program_id(1) == 0))
    def _():
        loss_ref[...] = jnp.zeros_like(loss_ref)
    loss_ref[...] += part


def _vq(z_e, codebook):
    B, H, W, C = z_e.shape
    zq, loss_sum = pl.pallas_call(
        _vq_kernel,
        grid=(B, H // _HT),
        in_specs=[
            pl.BlockSpec((1, _HT, W, C), lambda i, j: (i, j, 0, 0)),
            pl.BlockSpec(codebook.shape, lambda i, j: (0, 0)),
        ],
        out_specs=[
            pl.BlockSpec((1, _HT, W, C), lambda i, j: (i, j, 0, 0)),
            pl.BlockSpec((1, 1), lambda i, j: (0, 0)),
        ],
        out_shape=[
            jax.ShapeDtypeStruct(z_e.shape, jnp.float32),
            jax.ShapeDtypeStruct((1, 1), jnp.float32),
        ],
    )(z_e, codebook)
    mean_sq = loss_sum[0, 0] / (B * C * H * W)
    return zq, mean_sq


def kernel(x, enc_w1, enc_b1, enc_w2, enc_b2, enc_w3, enc_b3, codebook,
           dec_w1, dec_b1, dec_w2, dec_b2, dec_w3, dec_b3):
    commitment_cost = 0.25
    xh = jnp.transpose(x, (0, 2, 3, 1))
    h = jax.nn.relu(_conv_nhwc(xh, enc_w1, enc_b1, 2, 1))
    h = jax.nn.relu(_conv_nhwc(h, enc_w2, enc_b2, 2, 1))
    z_e = _conv_nhwc(h, enc_w3, enc_b3, 1, 1)

    z_q, mean_sq = _vq(z_e, codebook)
    commitment_loss = mean_sq
    codebook_loss = mean_sq
    vq_loss = codebook_loss + commitment_cost * commitment_loss

    g = jax.nn.relu(_conv_nhwc(z_q, dec_w1, dec_b1, 1, 1))
    g = jax.nn.relu(_convT_nhwc(g, dec_w2, dec_b2))
    x_recon = _convT_nhwc(g, dec_w3, dec_b3)
    return (jnp.transpose(x_recon, (0, 3, 1, 2)), vq_loss, commitment_loss, codebook_loss)
